# baseline (speedup 1.0000x reference)
import functools

import jax
import jax.numpy as jnp
from jax import lax
from jax.experimental import pallas as pl
from jax.experimental.pallas import tpu as pltpu
from jax.experimental.pallas import tpu_sc as plsc

N = 10000
NPAD = 10240
DIN = 128
DMODEL = 128
H = 4
DNB = 16
NW = 32
PT = NPAD // NW
GN = 16
NG = PT // GN
ROWS = GN * DNB
SCALE = 1999853.335557038

_f32 = jnp.float32
_i32 = jnp.int32


def _elu(x):
    return jnp.where(x > 0, x, jnp.exp(jnp.minimum(x, 0.0)) - 1.0)


_BLK = 2000
_GRID = N // _BLK


def _tc1_body(nf_ref, w_ref, b_ref, wv_ref, cc_ref, x_ref, v_ref, e_ref):
    x = jnp.dot(nf_ref[...], w_ref[...], preferred_element_type=_f32) + b_ref[...]
    x = _elu(x)
    x_ref[...] = x
    v_ref[...] = jnp.dot(x, wv_ref[...], preferred_element_type=_f32)
    e_ref[...] = jnp.dot(x, cc_ref[...], preferred_element_type=_f32)


def _tc1(nf, w, b, wv, cc):
    return pl.pallas_call(
        _tc1_body,
        grid=(_GRID,),
        in_specs=[
            pl.BlockSpec((_BLK, DIN), lambda i: (i, 0)),
            pl.BlockSpec((DIN, DMODEL), lambda i: (0, 0)),
            pl.BlockSpec((1, DMODEL), lambda i: (0, 0)),
            pl.BlockSpec((DMODEL, DMODEL), lambda i: (0, 0)),
            pl.BlockSpec((DMODEL, 2 * H), lambda i: (0, 0)),
        ],
        out_specs=[
            pl.BlockSpec((_BLK, DMODEL), lambda i: (i, 0)),
            pl.BlockSpec((_BLK, DMODEL), lambda i: (i, 0)),
            pl.BlockSpec((_BLK, 2 * H), lambda i: (i, 0)),
        ],
        out_shape=[
            jax.ShapeDtypeStruct((N, DMODEL), _f32),
            jax.ShapeDtypeStruct((N, DMODEL), _f32),
            jax.ShapeDtypeStruct((N, 2 * H), _f32),
        ],
    )(nf, w, b, wv, cc)


def _tc2_body(agg_ref, x_ref, lw_ref, lb_ref, g_ref, bb_ref, wv_ref, cc_ref,
              x2_ref, v_ref, e_ref):
    a = jnp.dot(agg_ref[...], lw_ref[...], preferred_element_type=_f32) + lb_ref[...]
    a = _elu(a)
    r = a + x_ref[...]
    mu = jnp.mean(r, axis=-1, keepdims=True)
    var = jnp.mean((r - mu) ** 2, axis=-1, keepdims=True)
    x2 = (r - mu) / jnp.sqrt(var + 1e-5) * g_ref[...] + bb_ref[...]
    x2_ref[...] = x2
    v_ref[...] = jnp.dot(x2, wv_ref[...], preferred_element_type=_f32)
    e_ref[...] = jnp.dot(x2, cc_ref[...], preferred_element_type=_f32)


def _tc2(agg, x, lw, lb, g, bb, wv, cc):
    return pl.pallas_call(
        _tc2_body,
        grid=(_GRID,),
        in_specs=[
            pl.BlockSpec((_BLK, DMODEL), lambda i: (i, 0)),
            pl.BlockSpec((_BLK, DMODEL), lambda i: (i, 0)),
            pl.BlockSpec((DMODEL, DMODEL), lambda i: (0, 0)),
            pl.BlockSpec((1, DMODEL), lambda i: (0, 0)),
            pl.BlockSpec((1, DMODEL), lambda i: (0, 0)),
            pl.BlockSpec((1, DMODEL), lambda i: (0, 0)),
            pl.BlockSpec((DMODEL, DMODEL), lambda i: (0, 0)),
            pl.BlockSpec((DMODEL, 2 * H), lambda i: (0, 0)),
        ],
        out_specs=[
            pl.BlockSpec((_BLK, DMODEL), lambda i: (i, 0)),
            pl.BlockSpec((_BLK, DMODEL), lambda i: (i, 0)),
            pl.BlockSpec((_BLK, 2 * H), lambda i: (i, 0)),
        ],
        out_shape=[
            jax.ShapeDtypeStruct((N, DMODEL), _f32),
            jax.ShapeDtypeStruct((N, DMODEL), _f32),
            jax.ShapeDtypeStruct((N, 2 * H), _f32),
        ],
    )(agg, x, lw, lb, g, bb, wv, cc)


def _tc3_body(q_ref, x2_ref, ag_ref, lw_ref, lb_ref, g_ref, bb_ref,
              w0_ref, b0_ref, w1_ref, b1_ref, w2_ref, b2_ref, out_ref):
    del q_ref
    x2 = x2_ref[0]
    ag = ag_ref[0]
    a = _elu(jnp.dot(ag, lw_ref[...], preferred_element_type=_f32) + lb_ref[...])
    r = a + x2
    mu = jnp.mean(r, axis=-1, keepdims=True)
    var = jnp.mean((r - mu) ** 2, axis=-1, keepdims=True)
    x3 = (r - mu) / jnp.sqrt(var + 1e-5) * g_ref[...] + bb_ref[...]
    h1 = _elu(jnp.dot(x3, w0_ref[...], preferred_element_type=_f32) + b0_ref[...])
    h2 = _elu(jnp.dot(h1, w1_ref[...], preferred_element_type=_f32) + b1_ref[...])
    h3 = _elu(jnp.dot(h2, w2_ref[...], preferred_element_type=_f32) + b2_ref[...])
    out_ref[...] = h3 * SCALE


def _tc3(q, x2r, ag2r, lw, lb, g, bb, w0, b0, w1, b1, w2, b2):
    grid_spec = pltpu.PrefetchScalarGridSpec(
        num_scalar_prefetch=1,
        grid=(1,),
        in_specs=[
            pl.BlockSpec((1, 1, DMODEL), lambda i, q: (q[0], 0, 0)),
            pl.BlockSpec((1, 1, DMODEL), lambda i, q: (q[0], 0, 0)),
            pl.BlockSpec((DMODEL, DMODEL), lambda i, q: (0, 0)),
            pl.BlockSpec((1, DMODEL), lambda i, q: (0, 0)),
            pl.BlockSpec((1, DMODEL), lambda i, q: (0, 0)),
            pl.BlockSpec((1, DMODEL), lambda i, q: (0, 0)),
            pl.BlockSpec((128, 128), lambda i, q: (0, 0)),
            pl.BlockSpec((1, 128), lambda i, q: (0, 0)),
            pl.BlockSpec((128, 64), lambda i, q: (0, 0)),
            pl.BlockSpec((1, 64), lambda i, q: (0, 0)),
            pl.BlockSpec((64, 32), lambda i, q: (0, 0)),
            pl.BlockSpec((1, 32), lambda i, q: (0, 0)),
        ],
        out_specs=pl.BlockSpec((1, 32), lambda i, q: (0, 0)),
    )
    return pl.pallas_call(
        _tc3_body,
        grid_spec=grid_spec,
        out_shape=jax.ShapeDtypeStruct((1, 32), _f32),
    )(q, x2r, ag2r, lw, lb, g, bb, w0, b0, w1, b1, w2, b2)


@functools.cache
def _make_sc_gat():
    mesh = plsc.VectorSubcoreMesh(core_axis_name="c", subcore_axis_name="s")
    return functools.partial(
        pl.kernel,
        mesh=mesh,
        compiler_params=pltpu.CompilerParams(needs_layout_passes=False),
        out_type=jax.ShapeDtypeStruct((NPAD, DMODEL), _f32),
        scratch_types=[
            pltpu.VMEM((H * NPAD,), _f32),
            pltpu.VMEM((H * PT,), _f32),
            pltpu.VMEM((PT * DNB,), _i32),
            pltpu.VMEM((ROWS, DMODEL), _f32),
            pltpu.VMEM((ROWS, DMODEL), _f32),
            pltpu.VMEM((GN, DMODEL), _f32),
            pltpu.VMEM((GN, DMODEL), _f32),
            pltpu.SemaphoreType.DMA,
            pltpu.SemaphoreType.DMA,
            pltpu.SemaphoreType.DMA,
            pltpu.SemaphoreType.DMA,
        ],
    )(_sc_gat_body)


def _sc_gat(v, et, aidx):
    return _make_sc_gat()(v, et, aidx)


def _sc_gat_body(v_hbm, et_hbm, aidx_hbm, agg_hbm,
                 sd_v, es_v, idx_v, st0, st1, out0, out1, sem0, sem1,
                 semo0, semo1):
    wid = lax.axis_index("s") * 2 + lax.axis_index("c")
    base = wid * PT
    ibase = wid * (PT * DNB)

    pltpu.sync_copy(et_hbm.at[pl.ds(H * NPAD, H * NPAD)], sd_v)
    for h in range(H):
        pltpu.sync_copy(et_hbm.at[pl.ds(h * NPAD + base, PT)],
                        es_v.at[pl.ds(h * PT, PT)])
    pltpu.sync_copy(aidx_hbm.at[pl.ds(ibase, PT * DNB)], idx_v)
    pltpu.async_copy(v_hbm.at[idx_v.at[pl.ds(0, ROWS)]], st0, sem0)
    pltpu.async_copy(v_hbm.at[idx_v.at[pl.ds(ROWS, ROWS)]], st1, sem1)

    def _process(g, st_ref, out_ref):
        def nbody(n, carry):
            cols = idx_v[pl.ds(g * ROWS + n * DNB, DNB)]
            rowb = n * DNB
            for h in range(H):
                sdk = plsc.load_gather(sd_v, [cols + h * NPAD])
                esn = plsc.load_gather(
                    es_v, [jnp.full((GN,), h * PT + g * GN + n, _i32)])
                e = esn + sdk
                e = jnp.where(e > 0, e, 0.2 * e)
                ex = jnp.exp(e)
                s = jnp.sum(ex)
                acc0 = jnp.zeros((GN,), _f32)
                acc1 = jnp.zeros((GN,), _f32)
                for k in range(DNB):
                    a = ex.at[jnp.full((GN,), k, _i32)].get(
                        mode='promise_in_bounds')
                    v0 = st_ref[rowb + k, pl.ds(h * 32, GN)]
                    v1 = st_ref[rowb + k, pl.ds(h * 32 + GN, GN)]
                    acc0 = acc0 + a * v0
                    acc1 = acc1 + a * v1
                invs = 1.0 / jnp.broadcast_to(s, (GN,))
                out_ref[n, pl.ds(h * 32, GN)] = acc0 * invs
                out_ref[n, pl.ds(h * 32 + GN, GN)] = acc1 * invs
            return carry

        lax.fori_loop(0, GN, nbody, 0)

    def body(i, carry):
        g0 = 2 * i
        g1 = 2 * i + 1
        pltpu.make_async_copy(v_hbm.at[pl.ds(0, ROWS)], st0, sem0).wait()

        @pl.when(i > 0)
        def _():
            pltpu.make_async_copy(
                out0, agg_hbm.at[pl.ds(0, GN)], semo0).wait()

        _process(g0, st0, out0)
        pltpu.async_copy(out0, agg_hbm.at[pl.ds(base + g0 * GN, GN)], semo0)

        @pl.when(g0 + 2 < NG)
        def _():
            pltpu.async_copy(
                v_hbm.at[idx_v.at[pl.ds((g0 + 2) * ROWS, ROWS)]], st0, sem0)

        pltpu.make_async_copy(v_hbm.at[pl.ds(0, ROWS)], st1, sem1).wait()

        @pl.when(i > 0)
        def _():
            pltpu.make_async_copy(
                out1, agg_hbm.at[pl.ds(0, GN)], semo1).wait()

        _process(g1, st1, out1)
        pltpu.async_copy(out1, agg_hbm.at[pl.ds(base + g1 * GN, GN)], semo1)

        @pl.when(g1 + 2 < NG)
        def _():
            pltpu.async_copy(
                v_hbm.at[idx_v.at[pl.ds((g1 + 2) * ROWS, ROWS)]], st1, sem1)

        return carry

    lax.fori_loop(0, NG // 2, body, 0)
    pltpu.make_async_copy(out0, agg_hbm.at[pl.ds(0, GN)], semo0).wait()
    pltpu.make_async_copy(out1, agg_hbm.at[pl.ds(0, GN)], semo1).wait()


def _prep_layer(lp):
    wv = jnp.transpose(lp['Wv'], (1, 0, 2)).reshape(DMODEL, H * 32)
    cs = jnp.einsum('hde,he->dh', lp['Wq'], lp['a_src'])
    cd = jnp.einsum('hde,he->dh', lp['Wq'], lp['a_dst'])
    cc = jnp.concatenate([cs, cd], axis=1)
    return wv, cc


def _pack_et(e):
    return jnp.pad(e.T, ((0, 0), (0, NPAD - N))).reshape(-1)


def kernel(node_features, query_idxs, masks, adj, sim_results, params):
    del masks, sim_results
    nf = node_features[0]
    adj0 = adj[0].astype(_i32)
    adjp = jnp.pad(adj0, ((0, NPAD - N), (0, 0)))
    aidx = adjp.reshape(-1)

    l1, l2 = params['layers']
    wv1, cc1 = _prep_layer(l1)
    wv2, cc2 = _prep_layer(l2)

    x, v1, e1 = _tc1(nf, params['init_W'], params['init_b'].reshape(1, -1),
                     wv1, cc1)
    agg1 = _sc_gat(v1, _pack_et(e1), aidx)[:N]
    x2, v2, e2 = _tc2(agg1, x, l1['lin_W'], l1['lin_b'].reshape(1, -1),
                      l1['ln_g'].reshape(1, -1), l1['ln_b'].reshape(1, -1),
                      wv2, cc2)
    agg2 = _sc_gat(v2, _pack_et(e2), aidx)[:N]

    (w0, b0), (w1, b1), (w2, b2) = params['final']
    out = _tc3(query_idxs.astype(_i32),
               x2.reshape(N, 1, DMODEL), agg2.reshape(N, 1, DMODEL),
               l2['lin_W'], l2['lin_b'].reshape(1, -1),
               l2['ln_g'].reshape(1, -1), l2['ln_b'].reshape(1, -1),
               w0, b0.reshape(1, -1), w1, b1.reshape(1, -1),
               w2, b2.reshape(1, -1))
    return out
